# baseline (speedup 1.0000x reference)
import math
import numpy as np
import jax
import jax.numpy as jnp
from jax import lax
from jax.experimental import pallas as pl
from jax.experimental.pallas import tpu as pltpu
from jax.experimental.pallas import tpu_sc as plsc

_N = 128
_NOUT = _N * (_N + 1) // 2
_B = 4096
_NW = 32
_BPW = _B // _NW
_NDIAG = _N // 16
_NMAIN = 8192
_NGRP = 8
_XW = 576
_NBUF = 2


def _grp_col(r: int) -> int:
    return 8 * r * (r + 1)


def _packed_index_table() -> np.ndarray:
    rows_l, cols_l = np.tril_indices(_N, k=-1)
    diag = np.arange(_N, dtype=np.int64)
    rows = np.concatenate([diag, rows_l.astype(np.int64)])
    cols = np.concatenate([diag, cols_l.astype(np.int64)])
    grp = rows >> 4
    brow = rows & 15
    bcol = 8 * grp * (grp + 1) + cols
    return ((brow << 16) | bcol).astype(np.int32)


def _tril_body(
    x_hbm, idx_hbm, out_hbm, idx_v, xbuf0, xbuf1, obuf0, obuf1, sem_in, sem_out
):
    wid = lax.axis_index("s") * 2 + lax.axis_index("c")
    base = wid * _BPW
    pltpu.sync_copy(idx_hbm, idx_v)
    sqrt2 = jnp.full((16,), math.sqrt(2.0), dtype=jnp.float32)
    xbufs = [xbuf0, xbuf1]
    obufs = [obuf0, obuf1]

    lanes = lax.iota(jnp.int32, 16)

    def start_loads(b, xb):
        for r in range(_NGRP):
            pltpu.async_copy(
                x_hbm.at[b, pl.ds(16 * r, 16), pl.ds(0, 16 * (r + 1))],
                xb.at[:, pl.ds(_grp_col(r), 16 * (r + 1))],
                sem_in,
            )

    def wait_loads(xb):
        for r in range(_NGRP):
            pltpu.make_async_copy(
                x_hbm.at[0, pl.ds(16 * r, 16), pl.ds(0, 16 * (r + 1))],
                xb.at[:, pl.ds(_grp_col(r), 16 * (r + 1))],
                sem_in,
            ).wait()

    def compute(xb, ob):
        for j in range(_NDIAG):
            ob[pl.ds(j * 16, 16)] = plsc.load_gather(
                xb, [lanes, lanes + (_grp_col(j) + 16 * j)]
            )

        def gather16(o):
            idx = idx_v[pl.ds(o, 16)]
            r = lax.shift_right_logical(idx, 16)
            c = lax.bitwise_and(idx, 0xFFFF)
            ob[pl.ds(o, 16)] = plsc.load_gather(xb, [r, c]) * sqrt2

        plsc.parallel_loop(_NDIAG * 16, _NMAIN, step=16, unroll=8)(gather16)
        for o in range(_NMAIN, _NOUT, 16):
            gather16(o)

    start_loads(base, xbufs[0])

    def outer(g2, carry):
        for p in range(_NBUF):
            g = g2 * _NBUF + p

            @pl.when(g + 1 < _BPW)
            def _():
                start_loads(base + g + 1, xbufs[(p + 1) % _NBUF])

            wait_loads(xbufs[p])

            @pl.when(g >= _NBUF)
            def _():
                pltpu.make_async_copy(
                    obufs[p], out_hbm.at[base], sem_out
                ).wait()

            compute(xbufs[p], obufs[p])
            pltpu.async_copy(obufs[p], out_hbm.at[base + g], sem_out)
        return carry

    lax.fori_loop(0, _BPW // _NBUF, outer, 0)
    for p in range(_NBUF):
        pltpu.make_async_copy(obufs[p], out_hbm.at[base], sem_out).wait()


@jax.jit
def kernel(X):
    idx = jnp.asarray(_packed_index_table())
    mesh = plsc.VectorSubcoreMesh(core_axis_name="c", subcore_axis_name="s")
    run = pl.kernel(
        _tril_body,
        mesh=mesh,
        out_type=jax.ShapeDtypeStruct((_B, _NOUT), jnp.float32),
        scratch_types=[
            pltpu.VMEM((_NOUT,), jnp.int32),
            pltpu.VMEM((16, _XW), jnp.float32),
            pltpu.VMEM((16, _XW), jnp.float32),
            pltpu.VMEM((_NOUT,), jnp.float32),
            pltpu.VMEM((_NOUT,), jnp.float32),
            pltpu.SemaphoreType.DMA,
            pltpu.SemaphoreType.DMA,
        ],
        compiler_params=pltpu.CompilerParams(
            needs_layout_passes=False, use_tc_tiling_on_sc=False
        ),
    )
    return run(X, idx)
